# baseline (speedup 1.0000x reference)
import functools

import jax
import jax.numpy as jnp
from jax import lax
from jax.experimental import pallas as pl
from jax.experimental.pallas import tpu as pltpu
from jax.experimental.pallas import tpu_sc as plsc

VOCAB = 100000
EMBED = 64
CHORD_START = 1000
CHORD_END = 5000
B, S = 4096, 50

TOKS = B * S
NC, NS, L = 2, 16, 16
NW = NC * NS
TPW = TOKS // NW
CHUNK = 128
NCHUNK = TPW // CHUNK
NBUF = 10
NROUND = NCHUNK // NBUF

NPAIR = TOKS // 2
PR128 = NPAIR // 128
POST_ROWS = 32


def _sc_gather_body(ids_hbm, table_hbm, out_hbm, ids_v, buf, gsem, ssem):
    wid = lax.axis_index("s") * NC + lax.axis_index("c")
    K = NBUF // 2

    pltpu.sync_copy(ids_hbm.at[wid], ids_v)

    for bslot in range(K):
        pltpu.async_copy(table_hbm.at[ids_v.at[bslot]], buf.at[bslot],
                         gsem.at[bslot])

    def do_round(r, carry):
        for bslot in range(NBUF):
            j = r * NBUF + bslot
            bb = buf.at[bslot]
            pslot = (bslot + K) % NBUF
            pbb = buf.at[pslot]
            pltpu.make_async_copy(table_hbm.at[ids_v.at[j]], bb,
                                  gsem.at[bslot]).wait()

            @pl.when(j >= K)
            def _drain():
                pltpu.make_async_copy(
                    pbb, out_hbm.at[pl.ds((wid * NCHUNK + j - K) * CHUNK,
                                          CHUNK)], ssem.at[pslot]).wait()

            @pl.when(j + K < NCHUNK)
            def _prefetch():
                pltpu.async_copy(table_hbm.at[ids_v.at[j + K]], pbb,
                                 gsem.at[pslot])

            pltpu.async_copy(bb, out_hbm.at[pl.ds((wid * NCHUNK + j) * CHUNK,
                                                  CHUNK)], ssem.at[bslot])
        return carry

    lax.fori_loop(0, NROUND, do_round, 0)

    for bslot in range(NBUF - K, NBUF):
        j = (NROUND - 1) * NBUF + bslot
        pltpu.make_async_copy(
            buf.at[bslot],
            out_hbm.at[pl.ds((wid * NCHUNK + j) * CHUNK, CHUNK)],
            ssem.at[bslot]).wait()


_sc_gather = functools.partial(
    pl.kernel,
    out_type=jax.ShapeDtypeStruct((TOKS, EMBED), jnp.float32),
    mesh=plsc.VectorSubcoreMesh(core_axis_name="c", subcore_axis_name="s"),
    compiler_params=pltpu.CompilerParams(use_tc_tiling_on_sc=False),
    scratch_types=[
        pltpu.VMEM((NCHUNK, CHUNK), jnp.int32),
        pltpu.VMEM((NBUF, CHUNK, EMBED), jnp.float32),
        pltpu.SemaphoreType.DMA((NBUF,)),
        pltpu.SemaphoreType.DMA((NBUF,)),
    ],
)(_sc_gather_body)


def _half_out(g, tid, rid, cid, rp, tpb, w1_ref):
    is_chord = (tid >= CHORD_START) & (tid <= CHORD_END)
    m = jax.lax.broadcast_in_dim(is_chord.astype(jnp.float32),
                                 (POST_ROWS, 128, EMBED), (0, 1))
    ridx = jax.lax.broadcasted_iota(jnp.int32, (POST_ROWS, 128, 16), 2)
    oh_r = (rid[:, :, None] == ridx).astype(jnp.float32)
    tidx = jax.lax.broadcasted_iota(jnp.int32, (POST_ROWS, 128, 8), 2)
    oh_t = (cid[:, :, None] == tidx).astype(jnp.float32)
    contrib = (
        jax.lax.dot_general(oh_r, rp, (((2,), (0,)), ((), ())),
                            preferred_element_type=jnp.float32)
        + jax.lax.dot_general(oh_t, tpb, (((2,), (0,)), ((), ())),
                              preferred_element_type=jnp.float32))
    proj = jax.lax.dot_general(g, w1_ref[:], (((2,), (0,)), ((), ())),
                               preferred_element_type=jnp.float32)
    return g + m * (proj + contrib - g)


def _tc_post_body(rows_ref, stk_ref, root_ref, type_ref, w1_ref, w2_ref,
                  w3_ref, b_ref, out_ref):
    rp = jax.lax.dot_general(root_ref[:], w2_ref[:], (((1,), (0,)), ((), ())),
                             preferred_element_type=jnp.float32)
    tpb = jax.lax.dot_general(type_ref[:], w3_ref[:], (((1,), (0,)), ((), ())),
                              preferred_element_type=jnp.float32) + b_ref[:]

    blk = rows_ref[:]
    g_e = blk[:, :, 0:EMBED]
    g_o = blk[:, :, EMBED:2 * EMBED]
    out_ref[:, :, 0:EMBED] = _half_out(
        g_e, stk_ref[0], stk_ref[1], stk_ref[2], rp, tpb, w1_ref)
    out_ref[:, :, EMBED:2 * EMBED] = _half_out(
        g_o, stk_ref[3], stk_ref[4], stk_ref[5], rp, tpb, w1_ref)


_tc_post = pl.pallas_call(
    _tc_post_body,
    grid=(PR128 // POST_ROWS,),
    in_specs=[
        pl.BlockSpec((POST_ROWS, 128, 2 * EMBED), lambda i: (i, 0, 0)),
        pl.BlockSpec((6, POST_ROWS, 128), lambda i: (0, i, 0)),
        pl.BlockSpec((16, EMBED), lambda i: (0, 0)),
        pl.BlockSpec((8, EMBED), lambda i: (0, 0)),
        pl.BlockSpec((EMBED, EMBED), lambda i: (0, 0)),
        pl.BlockSpec((EMBED, EMBED), lambda i: (0, 0)),
        pl.BlockSpec((EMBED, EMBED), lambda i: (0, 0)),
        pl.BlockSpec((1, EMBED), lambda i: (0, 0)),
    ],
    out_specs=pl.BlockSpec((POST_ROWS, 128, 2 * EMBED), lambda i: (i, 0, 0)),
    out_shape=jax.ShapeDtypeStruct((PR128, 128, 2 * EMBED), jnp.float32),
)


def kernel(token_ids, chord_root_ids, chord_type_ids, token_table, root_table,
           type_table, W, b):
    ids3d = token_ids.astype(jnp.int32).reshape(NW, NCHUNK, CHUNK)

    def eo(x):
        x2 = x.astype(jnp.int32).reshape(NPAIR, 2)
        return x2[:, 0].reshape(PR128, 128), x2[:, 1].reshape(PR128, 128)

    ids_e, ids_o = eo(token_ids)
    roots_e, roots_o = eo(chord_root_ids)
    types_e, types_o = eo(chord_type_ids)
    stacked = jnp.stack([ids_e, roots_e, types_e, ids_o, roots_o, types_o])

    w1t = lax.slice(W, (0, 0), (EMBED, EMBED)).T
    w2t = lax.slice(W, (0, EMBED), (EMBED, 2 * EMBED)).T
    w3t = lax.slice(W, (0, 2 * EMBED), (EMBED, 3 * EMBED)).T
    root_pad = jnp.pad(root_table, ((0, 16 - root_table.shape[0]), (0, 0)))

    rows = _sc_gather(ids3d, token_table).reshape(PR128, 128, 2 * EMBED)
    out = _tc_post(rows, stacked, root_pad, type_table, w1t, w2t, w3t,
                   b.reshape(1, EMBED))
    return out.reshape(B, S, EMBED)


# speedup vs baseline: 1.2035x; 1.2035x over previous
import functools

import jax
import jax.numpy as jnp
from jax import lax
from jax.experimental import pallas as pl
from jax.experimental.pallas import tpu as pltpu
from jax.experimental.pallas import tpu_sc as plsc

VOCAB = 100000
EMBED = 64
CHORD_START = 1000
CHORD_END = 5000
B, S = 4096, 50

TOKS = B * S
HTOKS = TOKS // 2
NC, NS, L = 2, 16, 16
NW = NC * NS
TPW = HTOKS // NW
CHUNK = 64
NCHUNK = TPW // CHUNK
NBUF = 10
NROUND = NCHUNK // NBUF
K = NBUF // 2

COMBO_ZROW = 104
COMBO_ROWS = 112

POST_ROWS = 32
HROWS128 = HTOKS // 128


def _tc_combo_body(root_ref, type_ref, w2_ref, w3_ref, b_ref, out_ref):
    rp = jax.lax.dot_general(root_ref[:], w2_ref[:], (((1,), (0,)), ((), ())),
                             preferred_element_type=jnp.float32)
    tp = jax.lax.dot_general(type_ref[:], w3_ref[:], (((1,), (0,)), ((), ())),
                             preferred_element_type=jnp.float32)
    tpb = tp + b_ref[:]
    for r in range(13):
        out_ref[8 * r:8 * r + 8, :] = rp[r:r + 1, :] + tpb
    out_ref[COMBO_ZROW:COMBO_ROWS, :] = jnp.zeros(
        (COMBO_ROWS - COMBO_ZROW, EMBED), jnp.float32)


_tc_combo = pl.pallas_call(
    _tc_combo_body,
    out_shape=jax.ShapeDtypeStruct((COMBO_ROWS, EMBED), jnp.float32),
)


def _sc_gather_body(ids_hbm, table_hbm, out_hbm, ids_v, buf, gsem, ssem):
    wid = lax.axis_index("s") * NC + lax.axis_index("c")

    pltpu.sync_copy(ids_hbm.at[wid], ids_v)

    for bslot in range(K):
        pltpu.async_copy(table_hbm.at[ids_v.at[bslot]], buf.at[bslot],
                         gsem.at[bslot])

    def do_round(r, carry):
        for bslot in range(NBUF):
            j = r * NBUF + bslot
            bb = buf.at[bslot]
            pslot = (bslot + K) % NBUF
            pbb = buf.at[pslot]
            pltpu.make_async_copy(table_hbm.at[ids_v.at[j]], bb,
                                  gsem.at[bslot]).wait()

            @pl.when(j >= K)
            def _drain():
                pltpu.make_async_copy(
                    pbb, out_hbm.at[pl.ds((wid * NCHUNK + j - K) * CHUNK,
                                          CHUNK)], ssem.at[pslot]).wait()

            @pl.when(j + K < NCHUNK)
            def _prefetch():
                pltpu.async_copy(table_hbm.at[ids_v.at[j + K]], pbb,
                                 gsem.at[pslot])

            pltpu.async_copy(bb, out_hbm.at[pl.ds((wid * NCHUNK + j) * CHUNK,
                                                  CHUNK)], ssem.at[bslot])
        return carry

    lax.fori_loop(0, NROUND, do_round, 0)

    for bslot in range(NBUF - K, NBUF):
        j = (NROUND - 1) * NBUF + bslot
        pltpu.make_async_copy(
            buf.at[bslot],
            out_hbm.at[pl.ds((wid * NCHUNK + j) * CHUNK, CHUNK)],
            ssem.at[bslot]).wait()


_sc_gather = functools.partial(
    pl.kernel,
    out_type=jax.ShapeDtypeStruct((HTOKS, EMBED), jnp.float32),
    mesh=plsc.VectorSubcoreMesh(core_axis_name="c", subcore_axis_name="s"),
    compiler_params=pltpu.CompilerParams(use_tc_tiling_on_sc=False),
    scratch_types=[
        pltpu.VMEM((NCHUNK, CHUNK), jnp.int32),
        pltpu.VMEM((NBUF, CHUNK, EMBED), jnp.float32),
        pltpu.SemaphoreType.DMA((NBUF,)),
        pltpu.SemaphoreType.DMA((NBUF,)),
    ],
)(_sc_gather_body)


def _tc_post_body(rows_ref, ids_ref, roots_ref, types_ref, combo_ref, w1_ref,
                  out_ref):
    g = rows_ref[:]
    tid = ids_ref[:]
    is_chord = (tid >= CHORD_START) & (tid <= CHORD_END)
    cidx = jnp.where(is_chord, roots_ref[:] * 8 + types_ref[:], COMBO_ZROW)
    kidx = jax.lax.broadcasted_iota(jnp.int32, (POST_ROWS, 128, COMBO_ROWS), 2)
    one_hot = (cidx[:, :, None] == kidx).astype(jnp.float32)
    contrib = jax.lax.dot_general(
        one_hot, combo_ref[:], (((2,), (0,)), ((), ())),
        preferred_element_type=jnp.float32)
    proj = jax.lax.dot_general(g, w1_ref[:], (((2,), (0,)), ((), ())),
                               preferred_element_type=jnp.float32)
    m = jax.lax.broadcast_in_dim(is_chord.astype(jnp.float32),
                                 (POST_ROWS, 128, EMBED), (0, 1))
    out_ref[:] = g + m * (proj + contrib - g)


_tc_post = pl.pallas_call(
    _tc_post_body,
    grid=(HROWS128 // POST_ROWS,),
    in_specs=[
        pl.BlockSpec((POST_ROWS, 128, EMBED), lambda i: (i, 0, 0)),
        pl.BlockSpec((POST_ROWS, 128), lambda i: (i, 0)),
        pl.BlockSpec((POST_ROWS, 128), lambda i: (i, 0)),
        pl.BlockSpec((POST_ROWS, 128), lambda i: (i, 0)),
        pl.BlockSpec((COMBO_ROWS, EMBED), lambda i: (0, 0)),
        pl.BlockSpec((EMBED, EMBED), lambda i: (0, 0)),
    ],
    out_specs=pl.BlockSpec((POST_ROWS, 128, EMBED), lambda i: (i, 0, 0)),
    out_shape=jax.ShapeDtypeStruct((HROWS128, 128, EMBED), jnp.float32),
)


def kernel(token_ids, chord_root_ids, chord_type_ids, token_table, root_table,
           type_table, W, b):
    ids = token_ids.astype(jnp.int32).reshape(TOKS)
    roots = chord_root_ids.astype(jnp.int32).reshape(TOKS)
    types = chord_type_ids.astype(jnp.int32).reshape(TOKS)

    w1t = lax.slice(W, (0, 0), (EMBED, EMBED)).T
    w2t = lax.slice(W, (0, EMBED), (EMBED, 2 * EMBED)).T
    w3t = lax.slice(W, (0, 2 * EMBED), (EMBED, 3 * EMBED)).T
    root_pad = jnp.pad(root_table, ((0, 16 - root_table.shape[0]), (0, 0)))

    combo = _tc_combo(root_pad, type_table, w2t, w3t, b.reshape(1, EMBED))

    halves = []
    for h in range(2):
        sl = slice(h * HTOKS, (h + 1) * HTOKS)
        ids3d = ids[sl].reshape(NW, NCHUNK, CHUNK)
        rows = _sc_gather(ids3d, token_table).reshape(HROWS128, 128, EMBED)
        out_h = _tc_post(rows,
                         ids[sl].reshape(HROWS128, 128),
                         roots[sl].reshape(HROWS128, 128),
                         types[sl].reshape(HROWS128, 128),
                         combo, w1t)
        halves.append(out_h.reshape(B // 2, S, EMBED))
    return jnp.concatenate(halves, axis=0)


# speedup vs baseline: 1.2762x; 1.0604x over previous
import functools

import jax
import jax.numpy as jnp
from jax import lax
from jax.experimental import pallas as pl
from jax.experimental.pallas import tpu as pltpu
from jax.experimental.pallas import tpu_sc as plsc

VOCAB = 100000
EMBED = 64
CHORD_START = 1000
CHORD_END = 5000
B, S = 4096, 50

TOKS = B * S
NC, NS, L = 2, 16, 16
NW = NC * NS
TPW = TOKS // NW
CHUNK = 128
NCHUNK = TPW // CHUNK
NBUF = 10
NROUND = NCHUNK // NBUF
K = NBUF // 2

COMBO_ZROW = 104
COMBO_ROWS = 112

POST_ROWS = 32


def _tc_combo_body(root_ref, type_ref, w2_ref, w3_ref, b_ref, out_ref):
    rp = jax.lax.dot_general(root_ref[:], w2_ref[:], (((1,), (0,)), ((), ())),
                             preferred_element_type=jnp.float32)
    tp = jax.lax.dot_general(type_ref[:], w3_ref[:], (((1,), (0,)), ((), ())),
                             preferred_element_type=jnp.float32)
    tpb = tp + b_ref[:]
    for r in range(13):
        out_ref[8 * r:8 * r + 8, :] = rp[r:r + 1, :] + tpb
    out_ref[COMBO_ZROW:COMBO_ROWS, :] = jnp.zeros(
        (COMBO_ROWS - COMBO_ZROW, EMBED), jnp.float32)


_tc_combo = pl.pallas_call(
    _tc_combo_body,
    out_shape=jax.ShapeDtypeStruct((COMBO_ROWS, EMBED), jnp.float32),
)


def _sc_gather_body(ids_hbm, table_hbm, out_hbm, ids_v, buf, gsem, ssem):
    wid = lax.axis_index("s") * NC + lax.axis_index("c")

    pltpu.sync_copy(ids_hbm.at[wid], ids_v)

    for bslot in range(K):
        pltpu.async_copy(table_hbm.at[ids_v.at[bslot]], buf.at[bslot],
                         gsem.at[bslot])

    def do_round(r, carry):
        for bslot in range(NBUF):
            j = r * NBUF + bslot
            bb = buf.at[bslot]
            pslot = (bslot + K) % NBUF
            pbb = buf.at[pslot]
            pltpu.make_async_copy(table_hbm.at[ids_v.at[j]], bb,
                                  gsem.at[bslot]).wait()

            @pl.when(j >= K)
            def _drain():
                pltpu.make_async_copy(
                    pbb, out_hbm.at[pl.ds((wid * NCHUNK + j - K) * CHUNK,
                                          CHUNK)], ssem.at[pslot]).wait()

            @pl.when(j + K < NCHUNK)
            def _prefetch():
                pltpu.async_copy(table_hbm.at[ids_v.at[j + K]], pbb,
                                 gsem.at[pslot])

            pltpu.async_copy(bb, out_hbm.at[pl.ds((wid * NCHUNK + j) * CHUNK,
                                                  CHUNK)], ssem.at[bslot])
        return carry

    lax.fori_loop(0, NROUND, do_round, 0)

    for bslot in range(NBUF - K, NBUF):
        j = (NROUND - 1) * NBUF + bslot
        pltpu.make_async_copy(
            buf.at[bslot],
            out_hbm.at[pl.ds((wid * NCHUNK + j) * CHUNK, CHUNK)],
            ssem.at[bslot]).wait()


_sc_gather = functools.partial(
    pl.kernel,
    out_type=jax.ShapeDtypeStruct((TOKS, EMBED), jnp.float32),
    mesh=plsc.VectorSubcoreMesh(core_axis_name="c", subcore_axis_name="s"),
    compiler_params=pltpu.CompilerParams(use_tc_tiling_on_sc=False),
    scratch_types=[
        pltpu.VMEM((NCHUNK, CHUNK), jnp.int32),
        pltpu.VMEM((NBUF, CHUNK, EMBED), jnp.float32),
        pltpu.SemaphoreType.DMA((NBUF,)),
        pltpu.SemaphoreType.DMA((NBUF,)),
    ],
)(_sc_gather_body)


def _tc_post_body(rows_ref, ids_ref, roots_ref, types_ref, combo_ref, w1_ref,
                  out_ref):
    g = rows_ref[:]
    tid = ids_ref[:]
    is_chord = (tid >= CHORD_START) & (tid <= CHORD_END)
    cidx = jnp.where(is_chord, roots_ref[:] * 8 + types_ref[:], COMBO_ZROW)
    kidx = jax.lax.broadcasted_iota(jnp.int32, (POST_ROWS, 128, COMBO_ROWS), 2)
    one_hot = (cidx[:, :, None] == kidx).astype(jnp.float32)
    contrib = jax.lax.dot_general(
        one_hot, combo_ref[:], (((2,), (0,)), ((), ())),
        preferred_element_type=jnp.float32)
    proj = jax.lax.dot_general(g, w1_ref[:], (((2,), (0,)), ((), ())),
                               preferred_element_type=jnp.float32)
    m = jax.lax.broadcast_in_dim(is_chord.astype(jnp.float32),
                                 (POST_ROWS, 128, EMBED), (0, 1))
    out_ref[:] = g + m * (proj + contrib - g)


_tc_post = pl.pallas_call(
    _tc_post_body,
    grid=(TOKS // (POST_ROWS * 128),),
    in_specs=[
        pl.BlockSpec((POST_ROWS, 128, EMBED), lambda i: (i, 0, 0)),
        pl.BlockSpec((POST_ROWS, 128), lambda i: (i, 0)),
        pl.BlockSpec((POST_ROWS, 128), lambda i: (i, 0)),
        pl.BlockSpec((POST_ROWS, 128), lambda i: (i, 0)),
        pl.BlockSpec((COMBO_ROWS, EMBED), lambda i: (0, 0)),
        pl.BlockSpec((EMBED, EMBED), lambda i: (0, 0)),
    ],
    out_specs=pl.BlockSpec((POST_ROWS, 128, EMBED), lambda i: (i, 0, 0)),
    out_shape=jax.ShapeDtypeStruct((TOKS // 128, 128, EMBED), jnp.float32),
)


def kernel(token_ids, chord_root_ids, chord_type_ids, token_table, root_table,
           type_table, W, b):
    ids3d = token_ids.astype(jnp.int32).reshape(NW, NCHUNK, CHUNK)
    ids2d = token_ids.astype(jnp.int32).reshape(TOKS // 128, 128)
    roots2d = chord_root_ids.astype(jnp.int32).reshape(TOKS // 128, 128)
    types2d = chord_type_ids.astype(jnp.int32).reshape(TOKS // 128, 128)

    w1t = lax.slice(W, (0, 0), (EMBED, EMBED)).T
    w2t = lax.slice(W, (0, EMBED), (EMBED, 2 * EMBED)).T
    w3t = lax.slice(W, (0, 2 * EMBED), (EMBED, 3 * EMBED)).T
    root_pad = jnp.pad(root_table, ((0, 16 - root_table.shape[0]), (0, 0)))

    combo = _tc_combo(root_pad, type_table, w2t, w3t, b.reshape(1, EMBED))
    rows = _sc_gather(ids3d, token_table).reshape(TOKS // 128, 128, EMBED)
    out = _tc_post(rows, ids2d, roots2d, types2d, combo, w1t)
    return out.reshape(B, S, EMBED)


# speedup vs baseline: 1.3177x; 1.0325x over previous
import functools

import jax
import jax.numpy as jnp
from jax import lax
from jax.experimental import pallas as pl
from jax.experimental.pallas import tpu as pltpu
from jax.experimental.pallas import tpu_sc as plsc

VOCAB = 100000
EMBED = 64
CHORD_START = 1000
CHORD_END = 5000
B, S = 4096, 50

TOKS = B * S
NC, NS, L = 2, 16, 16
NW = NC * NS
TPW = TOKS // NW
CHUNK = 128
NCHUNK = TPW // CHUNK
NBUF = 10
NROUND = NCHUNK // NBUF
K = NBUF // 2

COMBO_ZROW = 104
COMBO_ROWS = 112

POST_ROWS = 64


def _tc_combo_body(root_ref, type_ref, w2_ref, w3_ref, b_ref, out_ref):
    rp = jax.lax.dot_general(root_ref[:], w2_ref[:], (((1,), (0,)), ((), ())),
                             preferred_element_type=jnp.float32)
    tp = jax.lax.dot_general(type_ref[:], w3_ref[:], (((1,), (0,)), ((), ())),
                             preferred_element_type=jnp.float32)
    tpb = tp + b_ref[:]
    for r in range(13):
        out_ref[8 * r:8 * r + 8, :] = rp[r:r + 1, :] + tpb
    out_ref[COMBO_ZROW:COMBO_ROWS, :] = jnp.zeros(
        (COMBO_ROWS - COMBO_ZROW, EMBED), jnp.float32)


_tc_combo = pl.pallas_call(
    _tc_combo_body,
    out_shape=jax.ShapeDtypeStruct((COMBO_ROWS, EMBED), jnp.float32),
)


def _sc_gather_body(ids_hbm, table_hbm, out_hbm, ids_v, buf, gsem, ssem):
    wid = lax.axis_index("s") * NC + lax.axis_index("c")

    pltpu.sync_copy(ids_hbm.at[wid], ids_v)

    for bslot in range(K):
        pltpu.async_copy(table_hbm.at[ids_v.at[bslot]], buf.at[bslot],
                         gsem.at[bslot])

    def do_round(r, carry):
        for bslot in range(NBUF):
            j = r * NBUF + bslot
            bb = buf.at[bslot]
            pslot = (bslot + K) % NBUF
            pbb = buf.at[pslot]
            pltpu.make_async_copy(table_hbm.at[ids_v.at[j]], bb,
                                  gsem.at[bslot]).wait()

            @pl.when(j >= K)
            def _drain():
                pltpu.make_async_copy(
                    pbb, out_hbm.at[pl.ds((wid * NCHUNK + j - K) * CHUNK,
                                          CHUNK)], ssem.at[pslot]).wait()

            @pl.when(j + K < NCHUNK)
            def _prefetch():
                pltpu.async_copy(table_hbm.at[ids_v.at[j + K]], pbb,
                                 gsem.at[pslot])

            pltpu.async_copy(bb, out_hbm.at[pl.ds((wid * NCHUNK + j) * CHUNK,
                                                  CHUNK)], ssem.at[bslot])
        return carry

    lax.fori_loop(0, NROUND, do_round, 0)

    for bslot in range(NBUF - K, NBUF):
        j = (NROUND - 1) * NBUF + bslot
        pltpu.make_async_copy(
            buf.at[bslot],
            out_hbm.at[pl.ds((wid * NCHUNK + j) * CHUNK, CHUNK)],
            ssem.at[bslot]).wait()


_sc_gather = functools.partial(
    pl.kernel,
    out_type=jax.ShapeDtypeStruct((TOKS, EMBED), jnp.float32),
    mesh=plsc.VectorSubcoreMesh(core_axis_name="c", subcore_axis_name="s"),
    compiler_params=pltpu.CompilerParams(use_tc_tiling_on_sc=False),
    scratch_types=[
        pltpu.VMEM((NCHUNK, CHUNK), jnp.int32),
        pltpu.VMEM((NBUF, CHUNK, EMBED), jnp.float32),
        pltpu.SemaphoreType.DMA((NBUF,)),
        pltpu.SemaphoreType.DMA((NBUF,)),
    ],
)(_sc_gather_body)


def _tc_post_body(rows_ref, ids_ref, roots_ref, types_ref, combo_ref, w1_ref,
                  out_ref):
    g = rows_ref[:]
    tid = ids_ref[:]
    is_chord = (tid >= CHORD_START) & (tid <= CHORD_END)
    cidx = jnp.where(is_chord, roots_ref[:] * 8 + types_ref[:], COMBO_ZROW)
    kidx = jax.lax.broadcasted_iota(jnp.int32, (POST_ROWS, 128, COMBO_ROWS), 2)
    one_hot = (cidx[:, :, None] == kidx).astype(jnp.float32)
    contrib = jax.lax.dot_general(
        one_hot, combo_ref[:], (((2,), (0,)), ((), ())),
        preferred_element_type=jnp.float32)
    proj = jax.lax.dot_general(g, w1_ref[:], (((2,), (0,)), ((), ())),
                               preferred_element_type=jnp.float32)
    m = jax.lax.broadcast_in_dim(is_chord.astype(jnp.float32),
                                 (POST_ROWS, 128, EMBED), (0, 1))
    out_ref[:] = g + m * (proj + contrib - g)


_tc_post = pl.pallas_call(
    _tc_post_body,
    grid=(TOKS // (POST_ROWS * 128),),
    in_specs=[
        pl.BlockSpec((POST_ROWS, 128, EMBED), lambda i: (i, 0, 0)),
        pl.BlockSpec((POST_ROWS, 128), lambda i: (i, 0)),
        pl.BlockSpec((POST_ROWS, 128), lambda i: (i, 0)),
        pl.BlockSpec((POST_ROWS, 128), lambda i: (i, 0)),
        pl.BlockSpec((COMBO_ROWS, EMBED), lambda i: (0, 0)),
        pl.BlockSpec((EMBED, EMBED), lambda i: (0, 0)),
    ],
    out_specs=pl.BlockSpec((POST_ROWS, 128, EMBED), lambda i: (i, 0, 0)),
    out_shape=jax.ShapeDtypeStruct((TOKS // 128, 128, EMBED), jnp.float32),
)


def kernel(token_ids, chord_root_ids, chord_type_ids, token_table, root_table,
           type_table, W, b):
    ids3d = token_ids.astype(jnp.int32).reshape(NW, NCHUNK, CHUNK)
    ids2d = token_ids.astype(jnp.int32).reshape(TOKS // 128, 128)
    roots2d = chord_root_ids.astype(jnp.int32).reshape(TOKS // 128, 128)
    types2d = chord_type_ids.astype(jnp.int32).reshape(TOKS // 128, 128)

    w1t = lax.slice(W, (0, 0), (EMBED, EMBED)).T
    w2t = lax.slice(W, (0, EMBED), (EMBED, 2 * EMBED)).T
    w3t = lax.slice(W, (0, 2 * EMBED), (EMBED, 3 * EMBED)).T
    root_pad = jnp.pad(root_table, ((0, 16 - root_table.shape[0]), (0, 0)))

    combo = _tc_combo(root_pad, type_table, w2t, w3t, b.reshape(1, EMBED))
    rows = _sc_gather(ids3d, token_table).reshape(TOKS // 128, 128, EMBED)
    out = _tc_post(rows, ids2d, roots2d, types2d, combo, w1t)
    return out.reshape(B, S, EMBED)


# speedup vs baseline: 1.3376x; 1.0151x over previous
import functools

import jax
import jax.numpy as jnp
from jax import lax
from jax.experimental import pallas as pl
from jax.experimental.pallas import tpu as pltpu
from jax.experimental.pallas import tpu_sc as plsc

VOCAB = 100000
EMBED = 64
CHORD_START = 1000
CHORD_END = 5000
B, S = 4096, 50

TOKS = B * S
NC, NS, L = 2, 16, 16
NW = NC * NS
TPW = TOKS // NW
CHUNK = 128
NCHUNK = TPW // CHUNK
NBUF = 10
NROUND = NCHUNK // NBUF
K = NBUF // 2

COMBO_ZROW = 104
COMBO_ROWS = 112

POST_ROWS = 160


def _tc_combo_body(root_ref, type_ref, w2_ref, w3_ref, b_ref, out_ref):
    rp = jax.lax.dot_general(root_ref[:], w2_ref[:], (((1,), (0,)), ((), ())),
                             preferred_element_type=jnp.float32)
    tp = jax.lax.dot_general(type_ref[:], w3_ref[:], (((1,), (0,)), ((), ())),
                             preferred_element_type=jnp.float32)
    tpb = tp + b_ref[:]
    for r in range(13):
        out_ref[8 * r:8 * r + 8, :] = rp[r:r + 1, :] + tpb
    out_ref[COMBO_ZROW:COMBO_ROWS, :] = jnp.zeros(
        (COMBO_ROWS - COMBO_ZROW, EMBED), jnp.float32)


_tc_combo = pl.pallas_call(
    _tc_combo_body,
    out_shape=jax.ShapeDtypeStruct((COMBO_ROWS, EMBED), jnp.float32),
)


def _sc_gather_body(ids_hbm, table_hbm, out_hbm, ids_v, buf, gsem, ssem):
    wid = lax.axis_index("s") * NC + lax.axis_index("c")

    pltpu.sync_copy(ids_hbm.at[wid], ids_v)

    for bslot in range(K):
        pltpu.async_copy(table_hbm.at[ids_v.at[bslot]], buf.at[bslot],
                         gsem.at[bslot])

    def do_round(r, carry):
        for bslot in range(NBUF):
            j = r * NBUF + bslot
            bb = buf.at[bslot]
            pslot = (bslot + K) % NBUF
            pbb = buf.at[pslot]
            pltpu.make_async_copy(table_hbm.at[ids_v.at[j]], bb,
                                  gsem.at[bslot]).wait()

            @pl.when(j >= K)
            def _drain():
                pltpu.make_async_copy(
                    pbb, out_hbm.at[pl.ds((wid * NCHUNK + j - K) * CHUNK,
                                          CHUNK)], ssem.at[pslot]).wait()

            @pl.when(j + K < NCHUNK)
            def _prefetch():
                pltpu.async_copy(table_hbm.at[ids_v.at[j + K]], pbb,
                                 gsem.at[pslot])

            pltpu.async_copy(bb, out_hbm.at[pl.ds((wid * NCHUNK + j) * CHUNK,
                                                  CHUNK)], ssem.at[bslot])
        return carry

    lax.fori_loop(0, NROUND, do_round, 0)

    for bslot in range(NBUF - K, NBUF):
        j = (NROUND - 1) * NBUF + bslot
        pltpu.make_async_copy(
            buf.at[bslot],
            out_hbm.at[pl.ds((wid * NCHUNK + j) * CHUNK, CHUNK)],
            ssem.at[bslot]).wait()


_sc_gather = functools.partial(
    pl.kernel,
    out_type=jax.ShapeDtypeStruct((TOKS, EMBED), jnp.float32),
    mesh=plsc.VectorSubcoreMesh(core_axis_name="c", subcore_axis_name="s"),
    compiler_params=pltpu.CompilerParams(use_tc_tiling_on_sc=False),
    scratch_types=[
        pltpu.VMEM((NCHUNK, CHUNK), jnp.int32),
        pltpu.VMEM((NBUF, CHUNK, EMBED), jnp.float32),
        pltpu.SemaphoreType.DMA((NBUF,)),
        pltpu.SemaphoreType.DMA((NBUF,)),
    ],
)(_sc_gather_body)


def _tc_post_body(rows_ref, ids_ref, roots_ref, types_ref, combo_ref, w1_ref,
                  out_ref):
    g = rows_ref[:]
    tid = ids_ref[:]
    is_chord = (tid >= CHORD_START) & (tid <= CHORD_END)
    cidx = jnp.where(is_chord, roots_ref[:] * 8 + types_ref[:], COMBO_ZROW)
    kidx = jax.lax.broadcasted_iota(jnp.int32, (POST_ROWS, 128, COMBO_ROWS), 2)
    one_hot = (cidx[:, :, None] == kidx).astype(jnp.float32)
    contrib = jax.lax.dot_general(
        one_hot, combo_ref[:], (((2,), (0,)), ((), ())),
        preferred_element_type=jnp.float32)
    proj = jax.lax.dot_general(g, w1_ref[:], (((2,), (0,)), ((), ())),
                               preferred_element_type=jnp.float32)
    m = jax.lax.broadcast_in_dim(is_chord.astype(jnp.float32),
                                 (POST_ROWS, 128, EMBED), (0, 1))
    out_ref[:] = g + m * (proj + contrib - g)


_tc_post = pl.pallas_call(
    _tc_post_body,
    grid=(TOKS // (POST_ROWS * 128),),
    in_specs=[
        pl.BlockSpec((POST_ROWS, 128, EMBED), lambda i: (i, 0, 0)),
        pl.BlockSpec((POST_ROWS, 128), lambda i: (i, 0)),
        pl.BlockSpec((POST_ROWS, 128), lambda i: (i, 0)),
        pl.BlockSpec((POST_ROWS, 128), lambda i: (i, 0)),
        pl.BlockSpec((COMBO_ROWS, EMBED), lambda i: (0, 0)),
        pl.BlockSpec((EMBED, EMBED), lambda i: (0, 0)),
    ],
    out_specs=pl.BlockSpec((POST_ROWS, 128, EMBED), lambda i: (i, 0, 0)),
    out_shape=jax.ShapeDtypeStruct((TOKS // 128, 128, EMBED), jnp.float32),
)


def kernel(token_ids, chord_root_ids, chord_type_ids, token_table, root_table,
           type_table, W, b):
    ids3d = token_ids.astype(jnp.int32).reshape(NW, NCHUNK, CHUNK)
    ids2d = token_ids.astype(jnp.int32).reshape(TOKS // 128, 128)
    roots2d = chord_root_ids.astype(jnp.int32).reshape(TOKS // 128, 128)
    types2d = chord_type_ids.astype(jnp.int32).reshape(TOKS // 128, 128)

    w1t = lax.slice(W, (0, 0), (EMBED, EMBED)).T
    w2t = lax.slice(W, (0, EMBED), (EMBED, 2 * EMBED)).T
    w3t = lax.slice(W, (0, 2 * EMBED), (EMBED, 3 * EMBED)).T
    root_pad = jnp.pad(root_table, ((0, 16 - root_table.shape[0]), (0, 0)))

    combo = _tc_combo(root_pad, type_table, w2t, w3t, b.reshape(1, EMBED))
    rows = _sc_gather(ids3d, token_table).reshape(TOKS // 128, 128, EMBED)
    out = _tc_post(rows, ids2d, roots2d, types2d, combo, w1t)
    return out.reshape(B, S, EMBED)
